# Initial kernel scaffold; baseline (speedup 1.0000x reference)
import jax
import jax.numpy as jnp
from jax.experimental import pallas as pl


def kernel(x, edge_index, Wl1, bl1, Wr1, Wl2, bl2, Wr2, Wl3, bl3, Wr3, W_lin1, b_lin1, W_lin2, b_lin2):
    raise NotImplementedError("write your pallas kernel here")


# speedup vs baseline: 4.0840x; 4.0840x over previous
import functools

import jax
import jax.numpy as jnp
from jax import lax
from jax.experimental import pallas as pl
from jax.experimental.pallas import tpu as pltpu
from jax.experimental.pallas import tpu_sc as plsc

_N = 10000
_E = 160000
_D = 256
_HALF = 128
_B = 64
_NSUB = 16
_CHUNK = 80
_EPS = _E // _NSUB
_NCHUNK = _EPS // _CHUNK
_NPAD = 10240
_STRIPE = _NPAD // _NSUB

_sc_mesh = plsc.VectorSubcoreMesh(core_axis_name="c", subcore_axis_name="s",
                                  num_cores=2, num_subcores=16)


@functools.partial(
    pl.kernel,
    out_type=(
        jax.ShapeDtypeStruct((_NPAD, _HALF), jnp.float32),
        jax.ShapeDtypeStruct((_NPAD, _HALF), jnp.float32),
    ),
    mesh=_sc_mesh,
    scratch_types=[
        pltpu.VMEM((_NCHUNK, _CHUNK), jnp.int32),
        pltpu.VMEM((_NCHUNK, _CHUNK), jnp.int32),
        pltpu.VMEM((_CHUNK, _HALF), jnp.float32),
        pltpu.MemorySpace.VMEM_SHARED((_NPAD, _HALF), jnp.float32),
        pltpu.SemaphoreType.DMA,
    ],
)
def _sc_segsum(t_lo, t_hi, srcb, dstb, zrow_hbm, a_lo, a_hi,
               src_v, dst_v, rows_v, acc, sem):
    c = lax.axis_index("c")
    s = lax.axis_index("s")
    pltpu.sync_copy(zrow_hbm, acc.at[pl.ds(s * _STRIPE, _STRIPE)])
    pltpu.sync_copy(srcb.at[s], src_v)
    pltpu.sync_copy(dstb.at[s], dst_v)
    plsc.subcore_barrier()

    def run(tbl, out):
        def step(j, carry):
            pltpu.async_copy(tbl.at[src_v.at[j]], rows_v, sem).wait()
            pltpu.sync_copy(rows_v, acc.at[dst_v.at[j]], add=True)
            return carry

        lax.fori_loop(0, _NCHUNK, step, 0)
        plsc.subcore_barrier()
        pltpu.sync_copy(acc.at[pl.ds(s * _STRIPE, _STRIPE)],
                        out.at[pl.ds(s * _STRIPE, _STRIPE)])

    @pl.when(c == 0)
    def _():
        run(t_lo, a_lo)

    @pl.when(c == 1)
    def _():
        run(t_hi, a_hi)


_BLK = 1000


def _layer_body(al, ah, hl, hh, c0, wll, wlh, wrl, wrh, b, ol, oh):
    cnt = jnp.max(c0[...], axis=1, keepdims=True)
    inv = 1.0 / jnp.maximum(cnt, 1.0)
    acc = jnp.dot(al[...] * inv, wll[...], preferred_element_type=jnp.float32)
    acc = acc + jnp.dot(ah[...] * inv, wlh[...], preferred_element_type=jnp.float32)
    acc = acc + jnp.dot(hl[...], wrl[...], preferred_element_type=jnp.float32)
    acc = acc + jnp.dot(hh[...], wrh[...], preferred_element_type=jnp.float32)
    r = jnp.maximum(acc + b[...], 0.0)
    ol[...] = r[:, :_HALF]
    oh[...] = r[:, _HALF:]


def _tc_layer(a_lo, a_hi, h_lo, h_hi, cnt0, WlT, WrT, bl):
    row = lambda i: (i, 0)
    full = lambda i: (0, 0)
    return pl.pallas_call(
        _layer_body,
        grid=(_N // _BLK,),
        in_specs=[
            pl.BlockSpec((_BLK, _HALF), row),
            pl.BlockSpec((_BLK, _HALF), row),
            pl.BlockSpec((_BLK, _HALF), row),
            pl.BlockSpec((_BLK, _HALF), row),
            pl.BlockSpec((_BLK, _HALF), row),
            pl.BlockSpec((_HALF, _D), full),
            pl.BlockSpec((_HALF, _D), full),
            pl.BlockSpec((_HALF, _D), full),
            pl.BlockSpec((_HALF, _D), full),
            pl.BlockSpec((1, _D), full),
        ],
        out_specs=[
            pl.BlockSpec((_BLK, _HALF), row),
            pl.BlockSpec((_BLK, _HALF), row),
        ],
        out_shape=[
            jax.ShapeDtypeStruct((_N, _HALF), jnp.float32),
            jax.ShapeDtypeStruct((_N, _HALF), jnp.float32),
        ],
    )(a_lo, a_hi, h_lo, h_hi, cnt0,
      WlT[:_HALF], WlT[_HALF:], WrT[:_HALF], WrT[_HALF:], bl)


def _head_body(hl, hh, w1l, w1h, b1, w2, b2, out, pool_l, pool_h):
    def seg(b, carry):
        start = (625 * b + 3) // 4
        nxt = (625 * (b + 1) + 3) // 4
        ln = nxt - start
        idx = lax.broadcasted_iota(jnp.int32, (160, 1), 0)
        m = idx < ln
        rl = jnp.where(m, hl[pl.ds(start, 160), :], 0.0)
        rh = jnp.where(m, hh[pl.ds(start, 160), :], 0.0)
        pool_l[pl.ds(b, 1), :] = jnp.max(rl, axis=0, keepdims=True)
        pool_h[pl.ds(b, 1), :] = jnp.max(rh, axis=0, keepdims=True)
        return carry

    lax.fori_loop(0, _B, seg, 0)
    z = (jnp.dot(pool_l[...], w1l[...], preferred_element_type=jnp.float32)
         + jnp.dot(pool_h[...], w1h[...], preferred_element_type=jnp.float32)
         + b1[...])
    z = jnp.maximum(z, 0.0)
    zb = z.astype(jnp.bfloat16).astype(jnp.float32)
    wb = w2[...].astype(jnp.bfloat16).astype(jnp.float32)
    out[...] = jnp.sum(zb * wb, axis=1, keepdims=True) + b2[...]


def _tc_head(hl_pad, hh_pad, W1T, b1, w2row, b2):
    return pl.pallas_call(
        _head_body,
        out_shape=jax.ShapeDtypeStruct((_B, 1), jnp.float32),
        scratch_shapes=[
            pltpu.VMEM((_B, _HALF), jnp.float32),
            pltpu.VMEM((_B, _HALF), jnp.float32),
        ],
    )(hl_pad, hh_pad, W1T[:_HALF], W1T[_HALF:], b1, w2row, b2)


def kernel(x, edge_index, Wl1, bl1, Wr1, Wl2, bl2, Wr2, Wl3, bl3, Wr3,
           W_lin1, b_lin1, W_lin2, b_lin2):
    srcb = edge_index[0].reshape(_NSUB, _NCHUNK, _CHUNK)
    dstb = edge_index[1].reshape(_NSUB, _NCHUNK, _CHUNK)
    zrow = jnp.zeros((_STRIPE, _HALF), jnp.float32)
    ones_tbl = jnp.ones((_N, _HALF), jnp.float32)

    cnt0, _ = _sc_segsum(ones_tbl, ones_tbl, srcb, dstb, zrow)

    h_lo = x[:, :_HALF]
    h_hi = x[:, _HALF:]
    for Wl, bl, Wr in ((Wl1, bl1, Wr1), (Wl2, bl2, Wr2), (Wl3, bl3, Wr3)):
        a_lo, a_hi = _sc_segsum(h_lo, h_hi, srcb, dstb, zrow)
        h_lo, h_hi = _tc_layer(a_lo, a_hi, h_lo, h_hi, cnt0,
                               Wl.T, Wr.T, bl.reshape(1, _D))

    pad = ((0, 10016 - _N), (0, 0))
    out = _tc_head(jnp.pad(h_lo, pad), jnp.pad(h_hi, pad),
                   W_lin1.T, b_lin1.reshape(1, _HALF),
                   W_lin2, b_lin2.reshape(1, 1))
    return out[:, 0]


# speedup vs baseline: 6.1737x; 1.5117x over previous
import functools

import jax
import jax.numpy as jnp
from jax import lax
from jax.experimental import pallas as pl
from jax.experimental.pallas import tpu as pltpu
from jax.experimental.pallas import tpu_sc as plsc

_N = 10000
_E = 160000
_D = 256
_HALF = 128
_B = 64
_NSUB = 16
_CHUNK = 80
_EPS = _E // _NSUB
_NCHUNK = _EPS // _CHUNK
_NPAD = 10240
_STRIPE = _NPAD // _NSUB

_sc_mesh = plsc.VectorSubcoreMesh(core_axis_name="c", subcore_axis_name="s",
                                  num_cores=2, num_subcores=16)


@functools.partial(
    pl.kernel,
    out_type=(
        jax.ShapeDtypeStruct((_NPAD, _HALF), jnp.float32),
        jax.ShapeDtypeStruct((_NPAD, _HALF), jnp.float32),
    ),
    mesh=_sc_mesh,
    scratch_types=[
        pltpu.VMEM((_NCHUNK, _CHUNK), jnp.int32),
        pltpu.VMEM((_CHUNK, _HALF), jnp.float32),
        pltpu.MemorySpace.VMEM_SHARED((_NPAD, _HALF), jnp.float32),
    ],
)
def _sc_count(dstb, ones_hbm, zrow_hbm, cnt0, cnt1, dst_v, ones_v, acc):
    c = lax.axis_index("c")
    s = lax.axis_index("s")
    pltpu.sync_copy(zrow_hbm, acc.at[pl.ds(s * _STRIPE, _STRIPE)])
    pltpu.sync_copy(dstb.at[s], dst_v)
    pltpu.sync_copy(ones_hbm, ones_v)
    plsc.subcore_barrier()

    def run(lo, hi, out):
        def step(j, carry):
            pltpu.sync_copy(ones_v, acc.at[dst_v.at[j]], add=True)
            return carry

        lax.fori_loop(lo, hi, step, 0)
        plsc.subcore_barrier()
        pltpu.sync_copy(acc.at[pl.ds(s * _STRIPE, _STRIPE)],
                        out.at[pl.ds(s * _STRIPE, _STRIPE)])

    @pl.when(c == 0)
    def _():
        run(0, 62, cnt0)

    @pl.when(c == 1)
    def _():
        run(62, _NCHUNK, cnt1)


@functools.partial(
    pl.kernel,
    out_type=(
        jax.ShapeDtypeStruct((_NPAD, _HALF), jnp.float32),
        jax.ShapeDtypeStruct((_NPAD, _HALF), jnp.float32),
    ),
    mesh=_sc_mesh,
    scratch_types=[
        pltpu.VMEM((2, 1, _CHUNK), jnp.int32),
        pltpu.VMEM((_NCHUNK, _CHUNK), jnp.int32),
        pltpu.VMEM((_CHUNK, _HALF), jnp.float32),
        pltpu.VMEM((_CHUNK, _HALF), jnp.float32),
        pltpu.MemorySpace.VMEM_SHARED((_NPAD, _HALF), jnp.float32),
        pltpu.SemaphoreType.DMA,
        pltpu.SemaphoreType.DMA,
        pltpu.SemaphoreType.DMA,
        pltpu.SemaphoreType.DMA,
    ],
)
def _sc_segsum(t_lo, t_hi, srcb, dstb, zrow_hbm, a_lo, a_hi,
               sidx, dst_v, rows_a, rows_b, acc, sem_a, sem_b, sem_sa, sem_sb):
    c = lax.axis_index("c")
    s = lax.axis_index("s")
    pltpu.sync_copy(zrow_hbm, acc.at[pl.ds(s * _STRIPE, _STRIPE)])
    pltpu.sync_copy(dstb.at[s], dst_v)
    plsc.subcore_barrier()

    def run(tbl, out):
        last = _NCHUNK - 1
        pltpu.sync_copy(srcb.at[s, pl.ds(0, 1)], sidx.at[0])
        pltpu.async_copy(srcb.at[s, pl.ds(1, 1)], sidx.at[1], sem_sb)
        pltpu.async_copy(tbl.at[sidx.at[0, 0]], rows_a, sem_a)

        def step(k, carry):
            j0 = 2 * k
            pltpu.make_async_copy(srcb.at[s, pl.ds(j0 + 1, 1)], sidx.at[1], sem_sb).wait()
            pltpu.make_async_copy(tbl.at[sidx.at[0, 0]], rows_a, sem_a).wait()
            pltpu.async_copy(tbl.at[sidx.at[1, 0]], rows_b, sem_b)
            pltpu.async_copy(srcb.at[s, pl.ds(j0 + 2, 1)], sidx.at[0], sem_sa)
            pltpu.sync_copy(rows_a, acc.at[dst_v.at[j0]], add=True)
            pltpu.make_async_copy(srcb.at[s, pl.ds(j0 + 2, 1)], sidx.at[0], sem_sa).wait()
            pltpu.make_async_copy(tbl.at[sidx.at[1, 0]], rows_b, sem_b).wait()
            pltpu.async_copy(tbl.at[sidx.at[0, 0]], rows_a, sem_a)
            pltpu.async_copy(srcb.at[s, pl.ds(jnp.minimum(j0 + 3, last), 1)],
                             sidx.at[1], sem_sb)
            pltpu.sync_copy(rows_b, acc.at[dst_v.at[j0 + 1]], add=True)
            return carry

        lax.fori_loop(0, (_NCHUNK - 1) // 2, step, 0)
        pltpu.make_async_copy(srcb.at[s, pl.ds(last, 1)], sidx.at[1], sem_sb).wait()
        pltpu.make_async_copy(tbl.at[sidx.at[0, 0]], rows_a, sem_a).wait()
        pltpu.sync_copy(rows_a, acc.at[dst_v.at[last]], add=True)
        plsc.subcore_barrier()
        pltpu.sync_copy(acc.at[pl.ds(s * _STRIPE, _STRIPE)],
                        out.at[pl.ds(s * _STRIPE, _STRIPE)])

    @pl.when(c == 0)
    def _():
        run(t_lo, a_lo)

    @pl.when(c == 1)
    def _():
        run(t_hi, a_hi)


_BLK = 1000


def _layer_body(al, ah, hl, hh, c0, c1, wll, wlh, wrl, wrh, b, ol, oh):
    cnt = (jnp.max(c0[...], axis=1, keepdims=True)
           + jnp.max(c1[...], axis=1, keepdims=True))
    inv = 1.0 / jnp.maximum(cnt, 1.0)
    acc = jnp.dot(al[...] * inv, wll[...], preferred_element_type=jnp.float32)
    acc = acc + jnp.dot(ah[...] * inv, wlh[...], preferred_element_type=jnp.float32)
    acc = acc + jnp.dot(hl[...], wrl[...], preferred_element_type=jnp.float32)
    acc = acc + jnp.dot(hh[...], wrh[...], preferred_element_type=jnp.float32)
    r = jnp.maximum(acc + b[...], 0.0)
    ol[...] = r[:, :_HALF]
    oh[...] = r[:, _HALF:]


def _tc_layer(a_lo, a_hi, h_lo, h_hi, cnt0, cnt1, WlT, WrT, bl):
    row = lambda i: (i, 0)
    full = lambda i: (0, 0)
    return pl.pallas_call(
        _layer_body,
        grid=(_N // _BLK,),
        in_specs=[
            pl.BlockSpec((_BLK, _HALF), row),
            pl.BlockSpec((_BLK, _HALF), row),
            pl.BlockSpec((_BLK, _HALF), row),
            pl.BlockSpec((_BLK, _HALF), row),
            pl.BlockSpec((_BLK, _HALF), row),
            pl.BlockSpec((_BLK, _HALF), row),
            pl.BlockSpec((_HALF, _D), full),
            pl.BlockSpec((_HALF, _D), full),
            pl.BlockSpec((_HALF, _D), full),
            pl.BlockSpec((_HALF, _D), full),
            pl.BlockSpec((1, _D), full),
        ],
        out_specs=[
            pl.BlockSpec((_BLK, _HALF), row),
            pl.BlockSpec((_BLK, _HALF), row),
        ],
        out_shape=[
            jax.ShapeDtypeStruct((_N, _HALF), jnp.float32),
            jax.ShapeDtypeStruct((_N, _HALF), jnp.float32),
        ],
    )(a_lo, a_hi, h_lo, h_hi, cnt0, cnt1,
      WlT[:_HALF], WlT[_HALF:], WrT[:_HALF], WrT[_HALF:], bl)


def _head_body(hl, hh, w1l, w1h, b1, w2, b2, out, pool_l, pool_h):
    def seg(b, carry):
        start = (625 * b + 3) // 4
        nxt = (625 * (b + 1) + 3) // 4
        ln = nxt - start
        idx = lax.broadcasted_iota(jnp.int32, (160, 1), 0)
        m = idx < ln
        rl = jnp.where(m, hl[pl.ds(start, 160), :], 0.0)
        rh = jnp.where(m, hh[pl.ds(start, 160), :], 0.0)
        pool_l[pl.ds(b, 1), :] = jnp.max(rl, axis=0, keepdims=True)
        pool_h[pl.ds(b, 1), :] = jnp.max(rh, axis=0, keepdims=True)
        return carry

    lax.fori_loop(0, _B, seg, 0)
    z = (jnp.dot(pool_l[...], w1l[...], preferred_element_type=jnp.float32)
         + jnp.dot(pool_h[...], w1h[...], preferred_element_type=jnp.float32)
         + b1[...])
    z = jnp.maximum(z, 0.0)
    zb = z.astype(jnp.bfloat16).astype(jnp.float32)
    wb = w2[...].astype(jnp.bfloat16).astype(jnp.float32)
    out[...] = jnp.sum(zb * wb, axis=1, keepdims=True) + b2[...]


def _tc_head(hl_pad, hh_pad, W1T, b1, w2row, b2):
    return pl.pallas_call(
        _head_body,
        out_shape=jax.ShapeDtypeStruct((_B, 1), jnp.float32),
        scratch_shapes=[
            pltpu.VMEM((_B, _HALF), jnp.float32),
            pltpu.VMEM((_B, _HALF), jnp.float32),
        ],
    )(hl_pad, hh_pad, W1T[:_HALF], W1T[_HALF:], b1, w2row, b2)


def kernel(x, edge_index, Wl1, bl1, Wr1, Wl2, bl2, Wr2, Wl3, bl3, Wr3,
           W_lin1, b_lin1, W_lin2, b_lin2):
    srcb = edge_index[0].reshape(_NSUB, _NCHUNK, _CHUNK)
    dstb = edge_index[1].reshape(_NSUB, _NCHUNK, _CHUNK)
    zrow = jnp.zeros((_STRIPE, _HALF), jnp.float32)
    ones = jnp.ones((_CHUNK, _HALF), jnp.float32)

    cnt0, cnt1 = _sc_count(dstb, ones, zrow)

    h_lo = x[:, :_HALF]
    h_hi = x[:, _HALF:]
    for Wl, bl, Wr in ((Wl1, bl1, Wr1), (Wl2, bl2, Wr2), (Wl3, bl3, Wr3)):
        a_lo, a_hi = _sc_segsum(h_lo, h_hi, srcb, dstb, zrow)
        h_lo, h_hi = _tc_layer(a_lo, a_hi, h_lo, h_hi, cnt0, cnt1,
                               Wl.T, Wr.T, bl.reshape(1, _D))

    pad = ((0, 10016 - _N), (0, 0))
    out = _tc_head(jnp.pad(h_lo, pad), jnp.pad(h_hi, pad),
                   W_lin1.T, b_lin1.reshape(1, _HALF),
                   W_lin2, b_lin2.reshape(1, 1))
    return out[:, 0]
